# baseline (speedup 1.0000x reference)
import numpy as np

import jax
import jax.numpy as jnp
from jax.experimental import pallas as pl
from jax.experimental.pallas import tpu as pltpu

_BT = 512


def _placement1():
    P1 = np.zeros((3, 3, 96, 32), np.float32)
    for kh2 in range(3):
        for hp in range(2):
            for w in range(16):
                k = kh2 * 32 + hp * 16 + w
                for hp_ in range(2):
                    for wp_ in range(2):
                        for w2 in range(8):
                            n = hp_ * 16 + wp_ * 8 + w2
                            kh = 2 * kh2 + hp - hp_ - 1
                            kw = w - (2 * w2 + wp_) + 1
                            if 0 <= kh < 3 and 0 <= kw < 3:
                                P1[kh, kw, k, n] = 1.0
    return P1


def _placement2():
    P2 = np.zeros((3, 8, 8), np.float32)
    for kw in range(3):
        for w2 in range(8):
            w2p = w2 - kw + 1
            if 0 <= w2p < 8:
                P2[kw, w2, w2p] = 1.0
    return P2


_P1 = _placement1()
_P2 = _placement2()


def _body(x_ref, g1_ref, b1_ref, g2_ref, b2_ref, wfc_ref, bfc_ref, out_ref):
    x3 = jnp.transpose(x_ref[...], (1, 0, 2))
    bt = x3.shape[1]
    M = 8 * bt
    xb = x3.reshape(M, 32)
    zx = jnp.zeros((bt, 32), jnp.bfloat16)

    xu = jnp.concatenate([zx, xb[:-bt]], axis=0)
    xd = jnp.concatenate([xb[bt:], zx], axis=0)
    lhs1 = jnp.concatenate([xu, xb, xd], axis=1)
    z1 = jnp.dot(lhs1, g1_ref[...], preferred_element_type=jnp.float32)
    h1 = z1.astype(jnp.bfloat16)

    p = jnp.maximum(h1[:, :512], h1[:, 512:])
    p = jnp.maximum(p[:, :256], p[:, 256:])
    p = jnp.maximum(p + b1_ref[...], 0)

    zp = jnp.zeros((bt, 256), jnp.bfloat16)
    pu = jnp.concatenate([zp, p[:-bt]], axis=0)
    pd = jnp.concatenate([p[bt:], zp], axis=0)
    lhs2 = jnp.concatenate([pu, p, pd], axis=1)
    z2 = jnp.dot(lhs2, g2_ref[...], preferred_element_type=jnp.float32)
    h2a = jnp.maximum(z2 + b2_ref[...], 0.0)

    s = h2a[:, :256] + h2a[:, 256:]
    s = s[:, :128] + s[:, 128:]
    s = s[:, :64] + s[:, 64:]
    s = s[:4 * bt] + s[4 * bt:]
    s = s[:2 * bt] + s[2 * bt:]
    g = (s[:bt] + s[bt:]).astype(jnp.bfloat16)
    out_ref[...] = jnp.dot(g, wfc_ref[...],
                           preferred_element_type=jnp.float32) + bfc_ref[...]


def kernel(x, w1_oihw, b1, w2_oihw, b2, wfc, bfc):
    B = x.shape[0]
    nc = wfc.shape[0]
    x2 = x.astype(jnp.bfloat16).reshape(B, 8, 32)

    w1t = jnp.transpose(w1_oihw[:, 0], (1, 2, 0))
    g1 = jnp.einsum('hwkn,hwc->knc', _P1, w1t).reshape(96, 1024)
    g1 = g1.astype(jnp.bfloat16)
    w2t = jnp.transpose(w2_oihw, (2, 3, 1, 0))
    g2 = jnp.einsum('kab,jkco->jacbo', _P2, w2t).reshape(768, 512)
    g2 = g2.astype(jnp.bfloat16)
    b1t = jnp.tile(b1, 8).reshape(1, 256).astype(jnp.bfloat16)
    b2t = jnp.tile(b2, 8).reshape(1, 512)
    wfck = (wfc.T / 64.0).astype(jnp.bfloat16)
    bfct = bfc.reshape(1, nc)

    bt = _BT if B >= _BT else B
    Bp = -(-B // bt) * bt
    if Bp != B:
        x2 = jnp.pad(x2, ((0, Bp - B), (0, 0), (0, 0)))

    out = pl.pallas_call(
        _body,
        out_shape=jax.ShapeDtypeStruct((Bp, nc), jnp.float32),
        grid=(Bp // bt,),
        in_specs=[
            pl.BlockSpec((bt, 8, 32), lambda i: (i, 0, 0)),
            pl.BlockSpec((96, 1024), lambda i: (0, 0)),
            pl.BlockSpec((1, 256), lambda i: (0, 0)),
            pl.BlockSpec((768, 512), lambda i: (0, 0)),
            pl.BlockSpec((1, 512), lambda i: (0, 0)),
            pl.BlockSpec((64, nc), lambda i: (0, 0)),
            pl.BlockSpec((1, nc), lambda i: (0, 0)),
        ],
        out_specs=pl.BlockSpec((bt, nc), lambda i: (i, 0)),
        compiler_params=pltpu.CompilerParams(dimension_semantics=("parallel",)),
    )(x2, g1, b1t, g2, b2t, wfck, bfct)
    return out[:B]


# speedup vs baseline: 1.1156x; 1.1156x over previous
import numpy as np

import jax
import jax.numpy as jnp
from jax.experimental import pallas as pl
from jax.experimental.pallas import tpu as pltpu

_BT = 512


def _placement1():
    P1 = np.zeros((3, 3, 96, 32), np.float32)
    for kh2 in range(3):
        for hp in range(2):
            for w in range(16):
                k = kh2 * 32 + hp * 16 + w
                for hp_ in range(2):
                    for wp_ in range(2):
                        for w2 in range(8):
                            n = hp_ * 16 + wp_ * 8 + w2
                            kh = 2 * kh2 + hp - hp_ - 1
                            kw = w - (2 * w2 + wp_) + 1
                            if 0 <= kh < 3 and 0 <= kw < 3:
                                P1[kh, kw, k, n] = 1.0
    return P1


def _placement2():
    P2 = np.zeros((3, 8, 8), np.float32)
    for kw in range(3):
        for w2 in range(8):
            w2p = w2 - kw + 1
            if 0 <= w2p < 8:
                P2[kw, w2, w2p] = 1.0
    return P2


_P1 = _placement1()
_P2 = _placement2()


def _body(x_ref, g1_ref, b1_ref, g2_ref, b2_ref, wfc_ref, bfc_ref, out_ref):
    x3 = x_ref[...]
    bt = x3.shape[1]
    M = 8 * bt
    xb = x3.reshape(M, 32)
    zx = jnp.zeros((bt, 32), jnp.bfloat16)

    xu = jnp.concatenate([zx, xb[:-bt]], axis=0)
    xd = jnp.concatenate([xb[bt:], zx], axis=0)
    lhs1 = jnp.concatenate([xu, xb, xd], axis=1)
    z1 = jnp.dot(lhs1, g1_ref[...], preferred_element_type=jnp.float32)
    h1 = z1.astype(jnp.bfloat16)

    p = jnp.maximum(h1[:, :512], h1[:, 512:])
    p = jnp.maximum(p[:, :256], p[:, 256:])
    p = jnp.maximum(p + b1_ref[...], 0)

    zp = jnp.zeros((bt, 256), jnp.bfloat16)
    pu = jnp.concatenate([zp, p[:-bt]], axis=0)
    pd = jnp.concatenate([p[bt:], zp], axis=0)
    lhs2 = jnp.concatenate([pu, p, pd], axis=1)
    z2 = jnp.dot(lhs2, g2_ref[...], preferred_element_type=jnp.float32)
    h2a = jnp.maximum(z2 + b2_ref[...], 0.0)

    s = h2a[:, :256] + h2a[:, 256:]
    s = s[:, :128] + s[:, 128:]
    s = s[:, :64] + s[:, 64:]
    s = s[:4 * bt] + s[4 * bt:]
    s = s[:2 * bt] + s[2 * bt:]
    g = (s[:bt] + s[bt:]).astype(jnp.bfloat16)
    out_ref[...] = jnp.dot(g, wfc_ref[...],
                           preferred_element_type=jnp.float32) + bfc_ref[...]


def kernel(x, w1_oihw, b1, w2_oihw, b2, wfc, bfc):
    B = x.shape[0]
    nc = wfc.shape[0]
    x2 = x.astype(jnp.bfloat16).reshape(B, 8, 32).transpose(1, 0, 2)

    w1t = jnp.transpose(w1_oihw[:, 0], (1, 2, 0))
    g1 = jnp.einsum('hwkn,hwc->knc', _P1, w1t).reshape(96, 1024)
    g1 = g1.astype(jnp.bfloat16)
    w2t = jnp.transpose(w2_oihw, (2, 3, 1, 0))
    g2 = jnp.einsum('kab,jkco->jacbo', _P2, w2t).reshape(768, 512)
    g2 = g2.astype(jnp.bfloat16)
    b1t = jnp.tile(b1, 8).reshape(1, 256).astype(jnp.bfloat16)
    b2t = jnp.tile(b2, 8).reshape(1, 512)
    wfck = (wfc.T / 64.0).astype(jnp.bfloat16)
    bfct = bfc.reshape(1, nc)

    bt = _BT if B >= _BT else B
    Bp = -(-B // bt) * bt
    if Bp != B:
        x2 = jnp.pad(x2, ((0, 0), (0, Bp - B), (0, 0)))

    out = pl.pallas_call(
        _body,
        out_shape=jax.ShapeDtypeStruct((Bp, nc), jnp.float32),
        grid=(Bp // bt,),
        in_specs=[
            pl.BlockSpec((8, bt, 32), lambda i: (0, i, 0)),
            pl.BlockSpec((96, 1024), lambda i: (0, 0)),
            pl.BlockSpec((1, 256), lambda i: (0, 0)),
            pl.BlockSpec((768, 512), lambda i: (0, 0)),
            pl.BlockSpec((1, 512), lambda i: (0, 0)),
            pl.BlockSpec((64, nc), lambda i: (0, 0)),
            pl.BlockSpec((1, nc), lambda i: (0, 0)),
        ],
        out_specs=pl.BlockSpec((bt, nc), lambda i: (i, 0)),
        compiler_params=pltpu.CompilerParams(dimension_semantics=("parallel",)),
    )(x2, g1, b1t, g2, b2t, wfck, bfct)
    return out[:B]


# speedup vs baseline: 1.1482x; 1.0292x over previous
import numpy as np

import jax
import jax.numpy as jnp
from jax.experimental import pallas as pl
from jax.experimental.pallas import tpu as pltpu

_BT = 1024


def _placement1():
    P1 = np.zeros((3, 3, 96, 32), np.float32)
    for kh2 in range(3):
        for hp in range(2):
            for w in range(16):
                k = kh2 * 32 + hp * 16 + w
                for hp_ in range(2):
                    for wp_ in range(2):
                        for w2 in range(8):
                            n = hp_ * 16 + wp_ * 8 + w2
                            kh = 2 * kh2 + hp - hp_ - 1
                            kw = w - (2 * w2 + wp_) + 1
                            if 0 <= kh < 3 and 0 <= kw < 3:
                                P1[kh, kw, k, n] = 1.0
    return P1


def _placement2():
    P2 = np.zeros((3, 8, 8), np.float32)
    for kw in range(3):
        for w2 in range(8):
            w2p = w2 - kw + 1
            if 0 <= w2p < 8:
                P2[kw, w2, w2p] = 1.0
    return P2


_P1 = _placement1()
_P2 = _placement2()


def _body(x_ref, g1_ref, b1_ref, g2_ref, b2_ref, wfc_ref, bfc_ref, out_ref):
    x3 = x_ref[...]
    bt = x3.shape[1]
    M = 8 * bt
    xb = x3.reshape(M, 32)
    zx = jnp.zeros((bt, 32), jnp.bfloat16)

    xu = jnp.concatenate([zx, xb[:-bt]], axis=0)
    xd = jnp.concatenate([xb[bt:], zx], axis=0)
    lhs1 = jnp.concatenate([xu, xb, xd], axis=1)
    z1 = jnp.dot(lhs1, g1_ref[...], preferred_element_type=jnp.float32)
    h1 = z1.astype(jnp.bfloat16)

    p = jnp.maximum(h1[:, :512], h1[:, 512:])
    p = jnp.maximum(p[:, :256], p[:, 256:])
    p = jnp.maximum(p + b1_ref[...], 0)

    zp = jnp.zeros((bt, 256), jnp.bfloat16)
    pu = jnp.concatenate([zp, p[:-bt]], axis=0)
    pd = jnp.concatenate([p[bt:], zp], axis=0)
    lhs2 = jnp.concatenate([pu, p, pd], axis=1)
    z2 = jnp.dot(lhs2, g2_ref[...], preferred_element_type=jnp.float32)
    h2a = jnp.maximum(z2 + b2_ref[...], 0.0)

    s = h2a[:, :256] + h2a[:, 256:]
    s = s[:, :128] + s[:, 128:]
    s = s[:, :64] + s[:, 64:]
    s = s[:4 * bt] + s[4 * bt:]
    s = s[:2 * bt] + s[2 * bt:]
    g = (s[:bt] + s[bt:]).astype(jnp.bfloat16)
    out_ref[...] = jnp.dot(g, wfc_ref[...],
                           preferred_element_type=jnp.float32) + bfc_ref[...]


def kernel(x, w1_oihw, b1, w2_oihw, b2, wfc, bfc):
    B = x.shape[0]
    nc = wfc.shape[0]
    x2 = x.astype(jnp.bfloat16).reshape(B, 8, 32).transpose(1, 0, 2)

    w1t = jnp.transpose(w1_oihw[:, 0], (1, 2, 0))
    g1 = jnp.einsum('hwkn,hwc->knc', _P1, w1t).reshape(96, 1024)
    g1 = g1.astype(jnp.bfloat16)
    w2t = jnp.transpose(w2_oihw, (2, 3, 1, 0))
    g2 = jnp.einsum('kab,jkco->jacbo', _P2, w2t).reshape(768, 512)
    g2 = g2.astype(jnp.bfloat16)
    b1t = jnp.tile(b1, 8).reshape(1, 256).astype(jnp.bfloat16)
    b2t = jnp.tile(b2, 8).reshape(1, 512)
    wfck = (wfc.T / 64.0).astype(jnp.bfloat16)
    bfct = bfc.reshape(1, nc)

    bt = _BT if B >= _BT else B
    Bp = -(-B // bt) * bt
    if Bp != B:
        x2 = jnp.pad(x2, ((0, 0), (0, Bp - B), (0, 0)))

    out = pl.pallas_call(
        _body,
        out_shape=jax.ShapeDtypeStruct((Bp, nc), jnp.float32),
        grid=(Bp // bt,),
        in_specs=[
            pl.BlockSpec((8, bt, 32), lambda i: (0, i, 0)),
            pl.BlockSpec((96, 1024), lambda i: (0, 0)),
            pl.BlockSpec((1, 256), lambda i: (0, 0)),
            pl.BlockSpec((768, 512), lambda i: (0, 0)),
            pl.BlockSpec((1, 512), lambda i: (0, 0)),
            pl.BlockSpec((64, nc), lambda i: (0, 0)),
            pl.BlockSpec((1, nc), lambda i: (0, 0)),
        ],
        out_specs=pl.BlockSpec((bt, nc), lambda i: (i, 0)),
        compiler_params=pltpu.CompilerParams(dimension_semantics=("parallel",)),
    )(x2, g1, b1t, g2, b2t, wfck, bfct)
    return out[:B]
